# baseline (speedup 1.0000x reference)
import jax
import jax.numpy as jnp
from jax import lax
from jax.experimental import pallas as pl
from jax.experimental.pallas import tpu as pltpu
from jax.experimental.pallas import tpu_sc as plsc

F32 = jnp.float32
NC = 2
NS = 16
NW = NC * NS


def _mesh():
    return plsc.VectorSubcoreMesh(
        core_axis_name="c", subcore_axis_name="s", num_cores=NC, num_subcores=NS
    )


def _pair_gather(table, idx0, idx1):
    V, D = table.shape
    B = idx0.shape[0]
    per = B // NW
    G = 200
    nch = per // G
    assert per % G == 0 and per % 8 == 0

    def body(table_h, i0_h, i1_h, o0_h, o1_h, idx_v, buf, sem):
        wid = lax.axis_index("s") * NC + lax.axis_index("c")
        base = wid * per
        for idx_h, out_h in ((i0_h, o0_h), (i1_h, o1_h)):
            pltpu.sync_copy(idx_h.at[pl.ds(base, per)], idx_v)

            def step(k, _, out_h=out_h):
                pltpu.async_copy(
                    table_h.at[idx_v.at[pl.ds(k * G, G)]], buf, sem
                ).wait()
                pltpu.sync_copy(buf, out_h.at[pl.ds(base + k * G, G)])
                return 0

            lax.fori_loop(0, nch, step, 0)

    out = jax.ShapeDtypeStruct((B, D), table.dtype)
    return pl.kernel(
        body,
        out_type=(out, out),
        mesh=_mesh(),
        scratch_types=[
            pltpu.VMEM((per,), jnp.int32),
            pltpu.VMEM((G, D), table.dtype),
            pltpu.SemaphoreType.DMA,
        ],
    )(table, idx0, idx1)


SENT = 1 << 30


def _sc_bin(ids, n_out):
    T = ids.shape[0]
    share = T // NW
    OW = ((n_out + NW - 1) // NW + 7) // 8 * 8
    CAP = share + 64
    RCAP = share + 7 * 64
    iters = (share + 15) // 16
    assert share < (1 << 14) and OW << 14 < SENT and RCAP % 8 == 0

    def body(ids_h, bins_h, counts_h, idbuf, lb, ctmp):
        c = lax.axis_index("c")
        s = lax.axis_index("s")
        q = c * NS + s
        iota = lax.broadcasted_iota(jnp.int32, (16,), 0)
        sentv = jnp.full((16,), SENT, jnp.int32)
        pltpu.sync_copy(ids_h.at[pl.ds(q * share, share)],
                        idbuf.at[pl.ds(0, share)])

        for o in range(NW):
            rbase = (o * NW + q) * RCAP

            def scan(i, cnt, o=o):
                d = idbuf[pl.ds(i * 16, 16)]
                off = i * 16 + iota
                loc = d - o * OW
                m = (loc >= 0) & (loc < OW) & (off < share)
                mi = jnp.where(m, 1, 0).astype(jnp.int32)
                pos = cnt + plsc.cumsum(mi) - 1
                plsc.store_scatter(lb, [pos], (loc << 14) | off, mask=m)
                return cnt + jnp.sum(mi)

            cnt = lax.fori_loop(0, iters, scan, jnp.int32(0))
            for j in range(4):
                plsc.store_scatter(lb, [cnt + j * 16 + iota], sentv)
            nch = (cnt + 63) >> 6

            def fstep(k, _):
                pltpu.sync_copy(lb.at[pl.ds(k * 64, 64)],
                                bins_h.at[pl.ds(rbase + k * 64, 64)])
                return 0

            lax.fori_loop(0, nch, fstep, 0)
            ctmp[pl.ds(0, 16)] = jnp.full((16,), nch * 64, jnp.int32)
            pltpu.sync_copy(
                ctmp, counts_h.at[pl.ds((o * NW + q) * 16, 16)])

    return pl.kernel(
        body,
        out_type=(jax.ShapeDtypeStruct((NW * NW * RCAP,), jnp.int32),
                  jax.ShapeDtypeStruct((NW * NW * 16,), jnp.int32)),
        mesh=_mesh(),
        scratch_types=[
            pltpu.VMEM((share + 16,), jnp.int32),
            pltpu.VMEM((CAP,), jnp.int32),
            pltpu.VMEM((16,), jnp.int32),
        ],
        compiler_params=pltpu.CompilerParams(needs_layout_passes=False),
    )(ids)


def _sc_accum(msg, bins, counts, n_out, share):
    T, D = msg.shape
    OW = ((n_out + NW - 1) // NW + 7) // 8 * 8
    RCAP = share + 7 * 64
    WR = 312
    SW = (OW + WR - 1) // WR
    LASTW = OW - (SW - 1) * WR
    GC = 112
    assert LASTW % 8 == 0 and 0 < LASTW <= WR

    zeros = jnp.zeros((WR, D), F32)

    def body(msg_h, bins_h, counts_h, zeros_h, out_h,
             cbuf, ebuf, gidx, lwb, gbuf, win, sem):
        c = lax.axis_index("c")
        s = lax.axis_index("s")
        o = c * NS + s
        iota = lax.broadcasted_iota(jnp.int32, (16,), 0)
        pltpu.sync_copy(counts_h.at[pl.ds(o * NW * 16, NW * 16)], cbuf)
        for j in range(GC // 16):
            gidx[pl.ds(j * 16, 16)] = jnp.zeros((16,), jnp.int32)

        def flush(n):
            pltpu.async_copy(msg_h.at[gidx], gbuf, sem).wait()

            def radd(r, _):
                loc = lwb[pl.ds(r, 16)][0]
                for j in range(D // 16):
                    plsc.addupdate(win.at[loc, pl.ds(j * 16, 16)],
                                   gbuf[r, pl.ds(j * 16, 16)])
                return 0

            lax.fori_loop(0, n, radd, 0)

        def window(w, _):
            wlo = w * WR
            pltpu.sync_copy(zeros_h, win)

            def per_tile(t, ccnt):
                cv = cbuf[pl.ds(t * 16, 16)]
                cnt_t = cv[0]
                ebase = (o * NW + t) * RCAP

                def ck(k, ccnt):
                    pltpu.sync_copy(
                        bins_h.at[pl.ds(ebase + k * 64, 64)], ebuf)

                    def sixteen(i, ccnt):
                        e = ebuf[pl.ds(i * 16, 16)]
                        loc = e >> 14
                        off = e & 16383
                        lw_ = loc - wlo
                        m = (lw_ >= 0) & (lw_ < WR)
                        mi = jnp.where(m, 1, 0).astype(jnp.int32)
                        pos = ccnt + plsc.cumsum(mi) - 1
                        plsc.store_scatter(gidx, [pos], t * share + off,
                                           mask=m)
                        plsc.store_scatter(lwb, [pos], lw_, mask=m)
                        return ccnt + jnp.sum(mi)

                    ccnt = lax.fori_loop(0, 4, sixteen, ccnt)
                    return lax.cond(
                        ccnt >= GC - 64,
                        lambda n: (flush(n), jnp.int32(0))[1],
                        lambda n: n, ccnt)

                return lax.fori_loop(0, cnt_t >> 6, ck, ccnt)

            ccnt = lax.fori_loop(0, NW, per_tile, jnp.int32(0))

            @pl.when(ccnt > 0)
            def _():
                flush(ccnt)

            dbase = o * OW + wlo

            @pl.when(w < SW - 1)
            def _():
                pltpu.sync_copy(win, out_h.at[pl.ds(dbase, WR)])

            @pl.when(w == SW - 1)
            def _():
                pltpu.sync_copy(win.at[pl.ds(0, LASTW)],
                                out_h.at[pl.ds(dbase, LASTW)])
            return 0

        lax.fori_loop(0, SW, window, 0)

    return pl.kernel(
        body,
        out_type=jax.ShapeDtypeStruct((NW * OW, D), F32),
        mesh=_mesh(),
        scratch_types=[
            pltpu.VMEM((NW * 16,), jnp.int32),
            pltpu.VMEM((64,), jnp.int32),
            pltpu.VMEM((GC,), jnp.int32),
            pltpu.VMEM((GC + 16,), jnp.int32),
            pltpu.VMEM((GC, D), F32),
            pltpu.VMEM((WR, D), F32),
            pltpu.SemaphoreType.DMA,
        ],
        compiler_params=pltpu.CompilerParams(needs_layout_passes=False),
    )(msg, bins, counts, zeros)


def _scatter_add(msg, ids, n_out):
    T, D = msg.shape
    bins, counts = _sc_bin(ids, n_out)
    return _sc_accum(msg, bins, counts, n_out, T // NW)


def _tc_msg(rows, xs, wms, bm, wgs, bg):
    T = xs[0].shape[0]
    D = wms[0].shape[1]
    nx = len(xs)
    grid = (T // rows,)

    def body(*refs):
        x = [refs[i][...] for i in range(nx)]
        wm = [refs[nx + i][...] for i in range(nx)]
        bmr = refs[2 * nx][...]
        wg = [refs[2 * nx + 1 + i][...] for i in range(nx)]
        bgr = refs[3 * nx + 1][...]
        out = refs[3 * nx + 2]
        hm = bmr
        hg = bgr
        for xi, wmi, wgi in zip(x, wm, wg):
            hm = hm + jnp.dot(xi, wmi, preferred_element_type=F32)
            hg = hg + jnp.dot(xi, wgi, preferred_element_type=F32)
        out[...] = jax.nn.silu(hm) * jax.nn.sigmoid(hg)

    in_specs = (
        [pl.BlockSpec((rows, x.shape[1]), lambda i: (i, 0)) for x in xs]
        + [pl.BlockSpec(w.shape, lambda i: (0, 0)) for w in wms]
        + [pl.BlockSpec((1, D), lambda i: (0, 0))]
        + [pl.BlockSpec(w.shape, lambda i: (0, 0)) for w in wgs]
        + [pl.BlockSpec((1, D), lambda i: (0, 0))]
    )
    return pl.pallas_call(
        body,
        grid=grid,
        in_specs=in_specs,
        out_specs=pl.BlockSpec((rows, D), lambda i: (i, 0)),
        out_shape=jax.ShapeDtypeStruct((T, D), F32),
    )(*xs, *wms, bm.reshape(1, D), *wgs, bg.reshape(1, D))


def _tc_update(rows, x, a, w1, w2, b, g, be):
    N, D = x.shape
    grid = (N // rows,)

    def body(x_r, a_r, w1_r, w2_r, b_r, g_r, be_r, out_r):
        xv = x_r[...]
        h = (jnp.dot(xv, w1_r[...], preferred_element_type=F32)
             + jnp.dot(a_r[...], w2_r[...], preferred_element_type=F32)
             + b_r[...])
        mu = jnp.mean(h, axis=-1, keepdims=True)
        var = jnp.mean(h * h, axis=-1, keepdims=True) - mu * mu
        hn = (h - mu) * lax.rsqrt(var + 1e-5) * g_r[...] + be_r[...]
        out_r[...] = xv + jax.nn.silu(hn)

    return pl.pallas_call(
        body,
        grid=grid,
        in_specs=[
            pl.BlockSpec((rows, D), lambda i: (i, 0)),
            pl.BlockSpec((rows, D), lambda i: (i, 0)),
            pl.BlockSpec(w1.shape, lambda i: (0, 0)),
            pl.BlockSpec(w2.shape, lambda i: (0, 0)),
            pl.BlockSpec((1, D), lambda i: (0, 0)),
            pl.BlockSpec((1, D), lambda i: (0, 0)),
            pl.BlockSpec((1, D), lambda i: (0, 0)),
        ],
        out_specs=pl.BlockSpec((rows, D), lambda i: (i, 0)),
        out_shape=jax.ShapeDtypeStruct((N, D), F32),
    )(x, a, w1, w2, b.reshape(1, D), g.reshape(1, D), be.reshape(1, D))


def kernel(atoms, bonds, edge_index, triplets, angle_feat,
           W_bm, b_bm, W_bg, b_bg, W_bu, b_bu, g_bu, be_bu,
           W_am, b_am, W_ag, b_ag, W_au, b_au, g_au, be_au):
    D = bonds.shape[1]
    t0 = triplets[0]
    t1 = triplets[1]
    e0 = edge_index[0]
    e1 = edge_index[1]

    b_ij, b_kj = _pair_gather(bonds, t0, t1)
    msg = _tc_msg(
        640, [b_ij, b_kj, angle_feat],
        [W_bm[:D], W_bm[D:2 * D], W_bm[2 * D:]], b_bm,
        [W_bg[:D], W_bg[D:2 * D], W_bg[2 * D:]], b_bg)
    agg = _scatter_add(msg, t0, bonds.shape[0])[:bonds.shape[0]]
    bonds2 = _tc_update(640, bonds, agg,
                        W_bu[:D], W_bu[D:], b_bu, g_bu, be_bu)

    a_src, a_dst = _pair_gather(atoms, e0, e1)
    msg2 = _tc_msg(
        640, [a_src, a_dst, bonds2],
        [W_am[:D], W_am[D:2 * D], W_am[2 * D:]], b_am,
        [W_ag[:D], W_ag[D:2 * D], W_ag[2 * D:]], b_ag)
    agg2 = _scatter_add(msg2, e1, atoms.shape[0])[:atoms.shape[0]]
    atoms2 = _tc_update(1000, atoms, agg2,
                        W_au[:D], W_au[D:], b_au, g_au, be_au)

    return atoms2, bonds2


# speedup vs baseline: 1.0076x; 1.0076x over previous
import jax
import jax.numpy as jnp
from jax import lax
from jax.experimental import pallas as pl
from jax.experimental.pallas import tpu as pltpu
from jax.experimental.pallas import tpu_sc as plsc

F32 = jnp.float32
NC = 2
NS = 16
NW = NC * NS


def _mesh():
    return plsc.VectorSubcoreMesh(
        core_axis_name="c", subcore_axis_name="s", num_cores=NC, num_subcores=NS
    )


def _pair_gather(table, idx0, idx1):
    V, D = table.shape
    B = idx0.shape[0]
    per = B // NW
    G = 200
    nch = per // G
    assert per % G == 0 and per % 8 == 0

    def body(table_h, i0_h, i1_h, o0_h, o1_h, idx_v, buf, sem):
        wid = lax.axis_index("s") * NC + lax.axis_index("c")
        base = wid * per
        for idx_h, out_h in ((i0_h, o0_h), (i1_h, o1_h)):
            pltpu.sync_copy(idx_h.at[pl.ds(base, per)], idx_v)

            def step(k, _, out_h=out_h):
                pltpu.async_copy(
                    table_h.at[idx_v.at[pl.ds(k * G, G)]], buf, sem
                ).wait()
                pltpu.sync_copy(buf, out_h.at[pl.ds(base + k * G, G)])
                return 0

            lax.fori_loop(0, nch, step, 0)

    out = jax.ShapeDtypeStruct((B, D), table.dtype)
    return pl.kernel(
        body,
        out_type=(out, out),
        mesh=_mesh(),
        scratch_types=[
            pltpu.VMEM((per,), jnp.int32),
            pltpu.VMEM((G, D), table.dtype),
            pltpu.SemaphoreType.DMA,
        ],
    )(table, idx0, idx1)


SENT = 1 << 30


def _sc_bin(ids, n_out):
    T = ids.shape[0]
    share = T // NW
    OW = ((n_out + NW - 1) // NW + 7) // 8 * 8
    CAP = share + 64
    RCAP = share + 7 * 64
    iters = (share + 15) // 16
    assert share < (1 << 14) and OW << 14 < SENT and RCAP % 8 == 0

    def body(ids_h, bins_h, counts_h, idbuf, lb, ctmp):
        c = lax.axis_index("c")
        s = lax.axis_index("s")
        q = c * NS + s
        iota = lax.broadcasted_iota(jnp.int32, (16,), 0)
        sentv = jnp.full((16,), SENT, jnp.int32)
        pltpu.sync_copy(ids_h.at[pl.ds(q * share, share)],
                        idbuf.at[pl.ds(0, share)])

        for o in range(NW):
            rbase = (o * NW + q) * RCAP

            def scan(i, cnt, o=o):
                d = idbuf[pl.ds(i * 16, 16)]
                off = i * 16 + iota
                loc = d - o * OW
                m = (loc >= 0) & (loc < OW) & (off < share)
                mi = jnp.where(m, 1, 0).astype(jnp.int32)
                pos = cnt + plsc.cumsum(mi) - 1
                plsc.store_scatter(lb, [pos], (loc << 14) | off, mask=m)
                return cnt + jnp.sum(mi)

            cnt = lax.fori_loop(0, iters, scan, jnp.int32(0))
            for j in range(4):
                plsc.store_scatter(lb, [cnt + j * 16 + iota], sentv)
            nch = (cnt + 63) >> 6

            def fstep(k, _):
                pltpu.sync_copy(lb.at[pl.ds(k * 64, 64)],
                                bins_h.at[pl.ds(rbase + k * 64, 64)])
                return 0

            lax.fori_loop(0, nch, fstep, 0)
            ctmp[pl.ds(0, 16)] = jnp.full((16,), nch * 64, jnp.int32)
            pltpu.sync_copy(
                ctmp, counts_h.at[pl.ds((o * NW + q) * 16, 16)])

    return pl.kernel(
        body,
        out_type=(jax.ShapeDtypeStruct((NW * NW * RCAP,), jnp.int32),
                  jax.ShapeDtypeStruct((NW * NW * 16,), jnp.int32)),
        mesh=_mesh(),
        scratch_types=[
            pltpu.VMEM((share + 16,), jnp.int32),
            pltpu.VMEM((CAP,), jnp.int32),
            pltpu.VMEM((16,), jnp.int32),
        ],
        compiler_params=pltpu.CompilerParams(needs_layout_passes=False),
    )(ids)


def _sc_accum(msg, bins, counts, n_out, share):
    T, D = msg.shape
    OW = ((n_out + NW - 1) // NW + 7) // 8 * 8
    RCAP = share + 7 * 64
    WR = 312
    SW = (OW + WR - 1) // WR
    LASTW = OW - (SW - 1) * WR
    GC = 112
    assert LASTW % 8 == 0 and 0 < LASTW <= WR

    zeros = jnp.zeros((WR, D), F32)

    EC = 512

    def body(msg_h, bins_h, counts_h, zeros_h, out_h,
             cbuf, ebuf, tbuf, gidx, lwb, gbuf, win, sem):
        c = lax.axis_index("c")
        s = lax.axis_index("s")
        o = c * NS + s
        iota = lax.broadcasted_iota(jnp.int32, (16,), 0)
        pltpu.sync_copy(counts_h.at[pl.ds(o * NW * 16, NW * 16)], cbuf)
        for j in range(GC // 16):
            gidx[pl.ds(j * 16, 16)] = jnp.zeros((16,), jnp.int32)

        def flush(n):
            pltpu.async_copy(msg_h.at[gidx], gbuf, sem).wait()

            def radd(r, _):
                loc = lwb[pl.ds(r, 16)][0]
                for j in range(D // 16):
                    plsc.addupdate(win.at[loc, pl.ds(j * 16, 16)],
                                   gbuf[r, pl.ds(j * 16, 16)])
                return 0

            lax.fori_loop(0, n, radd, 0)

        def window(w, _):
            wlo = w * WR
            cps = [pltpu.async_copy(
                bins_h.at[pl.ds((o * NW + t) * RCAP, EC)],
                ebuf.at[pl.ds(t * EC, EC)], sem) for t in range(NW)]
            pltpu.sync_copy(zeros_h, win)
            for cp in cps:
                cp.wait()

            def per_tile(t, ccnt):
                cv = cbuf[pl.ds(t * 16, 16)]
                cnt_t = cv[0]
                ebase = (o * NW + t) * RCAP

                def sixteen(e, wlo, ccnt, t):
                    loc = e >> 14
                    off = e & 16383
                    lw_ = loc - wlo
                    m = (lw_ >= 0) & (lw_ < WR)
                    mi = jnp.where(m, 1, 0).astype(jnp.int32)
                    pos = ccnt + plsc.cumsum(mi) - 1
                    plsc.store_scatter(gidx, [pos], t * share + off,
                                       mask=m)
                    plsc.store_scatter(lwb, [pos], lw_, mask=m)
                    return ccnt + jnp.sum(mi)

                def spill(n):
                    return lax.cond(
                        n >= GC - 64,
                        lambda v: (flush(v), jnp.int32(0))[1],
                        lambda v: v, n)

                def ck_main(k, ccnt, t=t):
                    def six(i, ccnt):
                        e = ebuf[pl.ds(t * EC + k * 64 + i * 16, 16)]
                        return sixteen(e, wlo, ccnt, t)

                    return spill(lax.fori_loop(0, 4, six, ccnt))

                def ck_tail(k, ccnt, t=t):
                    pltpu.sync_copy(
                        bins_h.at[pl.ds(ebase + k * 64, 64)], tbuf)

                    def six(i, ccnt):
                        e = tbuf[pl.ds(i * 16, 16)]
                        return sixteen(e, wlo, ccnt, t)

                    return spill(lax.fori_loop(0, 4, six, ccnt))

                nch = cnt_t >> 6
                nmain = jnp.minimum(nch, EC >> 6)
                ccnt = lax.fori_loop(0, nmain, ck_main, ccnt)
                return lax.fori_loop(EC >> 6, nch, ck_tail, ccnt)

            ccnt = lax.fori_loop(0, NW, per_tile, jnp.int32(0))

            @pl.when(ccnt > 0)
            def _():
                flush(ccnt)

            dbase = o * OW + wlo

            @pl.when(w < SW - 1)
            def _():
                pltpu.sync_copy(win, out_h.at[pl.ds(dbase, WR)])

            @pl.when(w == SW - 1)
            def _():
                pltpu.sync_copy(win.at[pl.ds(0, LASTW)],
                                out_h.at[pl.ds(dbase, LASTW)])
            return 0

        lax.fori_loop(0, SW, window, 0)

    return pl.kernel(
        body,
        out_type=jax.ShapeDtypeStruct((NW * OW, D), F32),
        mesh=_mesh(),
        scratch_types=[
            pltpu.VMEM((NW * 16,), jnp.int32),
            pltpu.VMEM((NW * EC,), jnp.int32),
            pltpu.VMEM((64,), jnp.int32),
            pltpu.VMEM((GC,), jnp.int32),
            pltpu.VMEM((GC + 16,), jnp.int32),
            pltpu.VMEM((GC, D), F32),
            pltpu.VMEM((WR, D), F32),
            pltpu.SemaphoreType.DMA,
        ],
        compiler_params=pltpu.CompilerParams(needs_layout_passes=False),
    )(msg, bins, counts, zeros)


def _scatter_add(msg, ids, n_out):
    T, D = msg.shape
    bins, counts = _sc_bin(ids, n_out)
    return _sc_accum(msg, bins, counts, n_out, T // NW)


def _tc_msg(rows, xs, wms, bm, wgs, bg):
    T = xs[0].shape[0]
    D = wms[0].shape[1]
    nx = len(xs)
    grid = (T // rows,)

    def body(*refs):
        x = [refs[i][...] for i in range(nx)]
        wm = [refs[nx + i][...] for i in range(nx)]
        bmr = refs[2 * nx][...]
        wg = [refs[2 * nx + 1 + i][...] for i in range(nx)]
        bgr = refs[3 * nx + 1][...]
        out = refs[3 * nx + 2]
        hm = bmr
        hg = bgr
        for xi, wmi, wgi in zip(x, wm, wg):
            hm = hm + jnp.dot(xi, wmi, preferred_element_type=F32)
            hg = hg + jnp.dot(xi, wgi, preferred_element_type=F32)
        out[...] = jax.nn.silu(hm) * jax.nn.sigmoid(hg)

    in_specs = (
        [pl.BlockSpec((rows, x.shape[1]), lambda i: (i, 0)) for x in xs]
        + [pl.BlockSpec(w.shape, lambda i: (0, 0)) for w in wms]
        + [pl.BlockSpec((1, D), lambda i: (0, 0))]
        + [pl.BlockSpec(w.shape, lambda i: (0, 0)) for w in wgs]
        + [pl.BlockSpec((1, D), lambda i: (0, 0))]
    )
    return pl.pallas_call(
        body,
        grid=grid,
        in_specs=in_specs,
        out_specs=pl.BlockSpec((rows, D), lambda i: (i, 0)),
        out_shape=jax.ShapeDtypeStruct((T, D), F32),
    )(*xs, *wms, bm.reshape(1, D), *wgs, bg.reshape(1, D))


def _tc_update(rows, x, a, w1, w2, b, g, be):
    N, D = x.shape
    grid = (N // rows,)

    def body(x_r, a_r, w1_r, w2_r, b_r, g_r, be_r, out_r):
        xv = x_r[...]
        h = (jnp.dot(xv, w1_r[...], preferred_element_type=F32)
             + jnp.dot(a_r[...], w2_r[...], preferred_element_type=F32)
             + b_r[...])
        mu = jnp.mean(h, axis=-1, keepdims=True)
        var = jnp.mean(h * h, axis=-1, keepdims=True) - mu * mu
        hn = (h - mu) * lax.rsqrt(var + 1e-5) * g_r[...] + be_r[...]
        out_r[...] = xv + jax.nn.silu(hn)

    return pl.pallas_call(
        body,
        grid=grid,
        in_specs=[
            pl.BlockSpec((rows, D), lambda i: (i, 0)),
            pl.BlockSpec((rows, D), lambda i: (i, 0)),
            pl.BlockSpec(w1.shape, lambda i: (0, 0)),
            pl.BlockSpec(w2.shape, lambda i: (0, 0)),
            pl.BlockSpec((1, D), lambda i: (0, 0)),
            pl.BlockSpec((1, D), lambda i: (0, 0)),
            pl.BlockSpec((1, D), lambda i: (0, 0)),
        ],
        out_specs=pl.BlockSpec((rows, D), lambda i: (i, 0)),
        out_shape=jax.ShapeDtypeStruct((N, D), F32),
    )(x, a, w1, w2, b.reshape(1, D), g.reshape(1, D), be.reshape(1, D))


def kernel(atoms, bonds, edge_index, triplets, angle_feat,
           W_bm, b_bm, W_bg, b_bg, W_bu, b_bu, g_bu, be_bu,
           W_am, b_am, W_ag, b_ag, W_au, b_au, g_au, be_au):
    D = bonds.shape[1]
    t0 = triplets[0]
    t1 = triplets[1]
    e0 = edge_index[0]
    e1 = edge_index[1]

    b_ij, b_kj = _pair_gather(bonds, t0, t1)
    msg = _tc_msg(
        640, [b_ij, b_kj, angle_feat],
        [W_bm[:D], W_bm[D:2 * D], W_bm[2 * D:]], b_bm,
        [W_bg[:D], W_bg[D:2 * D], W_bg[2 * D:]], b_bg)
    agg = _scatter_add(msg, t0, bonds.shape[0])[:bonds.shape[0]]
    bonds2 = _tc_update(640, bonds, agg,
                        W_bu[:D], W_bu[D:], b_bu, g_bu, be_bu)

    a_src, a_dst = _pair_gather(atoms, e0, e1)
    msg2 = _tc_msg(
        640, [a_src, a_dst, bonds2],
        [W_am[:D], W_am[D:2 * D], W_am[2 * D:]], b_am,
        [W_ag[:D], W_ag[D:2 * D], W_ag[2 * D:]], b_ag)
    agg2 = _scatter_add(msg2, e1, atoms.shape[0])[:atoms.shape[0]]
    atoms2 = _tc_update(1000, atoms, agg2,
                        W_au[:D], W_au[D:], b_au, g_au, be_au)

    return atoms2, bonds2
